# baseline (speedup 1.0000x reference)
import functools

import jax
import jax.numpy as jnp
import numpy as np
from jax import lax
from jax.experimental import pallas as pl
from jax.experimental.pallas import tpu as pltpu
from jax.experimental.pallas import tpu_sc as plsc

B = 16384
D = 32
NW = 32
BPW = B // NW
CHUNK = 128


def _gather_body(ut_hbm, it_hbm, iu_hbm, ii_hbm, u_out, i_out,
                 iu_v, ii_v, rows, sem):
    wid = lax.axis_index("s") * 2 + lax.axis_index("c")
    base = wid * BPW
    pltpu.sync_copy(iu_hbm.at[pl.ds(base, BPW)], iu_v)
    pltpu.sync_copy(ii_hbm.at[pl.ds(base, BPW)], ii_v)
    for idx_v, out in ((iu_v, u_out), (ii_v, i_out)):
        copies = []
        for j in range(BPW // CHUNK):
            s = pl.ds(j * CHUNK, CHUNK)
            copies.append(pltpu.async_copy(ut_hbm.at[idx_v.at[s]]
                                           if out is u_out
                                           else it_hbm.at[idx_v.at[s]],
                                           rows.at[s], sem))
        for c in copies:
            c.wait()
        pltpu.sync_copy(rows, out.at[pl.ds(base, BPW)])


@jax.jit
def _sc_gather(u128, i128, idx_u4, idx_i4):
    mesh = plsc.VectorSubcoreMesh(core_axis_name="c", subcore_axis_name="s")
    out = jax.ShapeDtypeStruct((B, 128), jnp.float32)
    return pl.kernel(
        _gather_body,
        mesh=mesh,
        out_type=(out, out),
        scratch_types=[
            pltpu.VMEM((BPW,), jnp.int32),
            pltpu.VMEM((BPW,), jnp.int32),
            pltpu.VMEM((BPW, 128), jnp.float32),
            pltpu.SemaphoreType.DMA,
        ],
    )(u128, i128, idx_u4, idx_i4)


TB = 4096


def _select4(h, o):
    acc = jnp.where(o == 0, h[:, 0:64], 0.0)
    for c in range(1, 4):
        acc = acc + jnp.where(o == c, h[:, 64 * c:64 * (c + 1)], 0.0)
    return acc


def _mlp_body(u_ref, i_ref, ou_ref, oi_ref, w1u_ref, w1i_ref, b1_ref,
              w2_ref, b2_ref, w3_ref, b3_ref, o_ref):
    hu = jnp.dot(u_ref[...], w1u_ref[...], preferred_element_type=jnp.float32)
    hi = jnp.dot(i_ref[...], w1i_ref[...], preferred_element_type=jnp.float32)
    h = _select4(hu, ou_ref[...]) + _select4(hi, oi_ref[...]) + b1_ref[...]
    h = jax.nn.relu(h)
    h = jax.nn.relu(
        jnp.dot(h, w2_ref[...], preferred_element_type=jnp.float32) + b2_ref[...])
    o_ref[...] = (
        jnp.dot(h, w3_ref[...], preferred_element_type=jnp.float32) + b3_ref[...])


@jax.jit
def _tc_mlp(u_g, i_g, o_u, o_i, W1su, W1si, b1, W2, b2, W3, b3):
    full = lambda r, c: pl.BlockSpec((r, c), lambda i: (0, 0))
    return pl.pallas_call(
        _mlp_body,
        grid=(B // TB,),
        in_specs=[
            pl.BlockSpec((TB, 128), lambda i: (i, 0)),
            pl.BlockSpec((TB, 128), lambda i: (i, 0)),
            pl.BlockSpec((TB, 1), lambda i: (i, 0)),
            pl.BlockSpec((TB, 1), lambda i: (i, 0)),
            full(128, 256), full(128, 256), full(1, 64),
            full(64, 16), full(1, 16),
            full(16, 1), full(1, 1),
        ],
        out_specs=pl.BlockSpec((TB, 1), lambda i: (i, 0)),
        out_shape=jax.ShapeDtypeStruct((B, 1), jnp.float32),
    )(u_g, i_g, o_u, o_i, W1su, W1si, b1, W2, b2, W3, b3)


def _stack4(W):
    Z = jnp.zeros((D, 64), W.dtype)
    cols = []
    for c in range(4):
        blocks = [Z] * 4
        blocks[c] = W
        cols.append(jnp.concatenate(blocks, axis=0))
    return jnp.concatenate(cols, axis=1)


def kernel(inputs, user_table, item_table, W1, b1, W2, b2, W3, b3):
    idx_u = inputs[:, 0]
    idx_i = inputs[:, 1]
    u128 = user_table.reshape(250000, 128)
    i128 = item_table.reshape(250000, 128)
    u_g, i_g = _sc_gather(u128, i128, idx_u >> 2, idx_i >> 2)
    return _tc_mlp(
        u_g, i_g,
        (idx_u & 3).reshape(B, 1), (idx_i & 3).reshape(B, 1),
        _stack4(W1[:D, :]), _stack4(W1[D:, :]), b1.reshape(1, 64),
        W2, b2.reshape(1, 16),
        W3, b3.reshape(1, 1),
    )


# speedup vs baseline: 3.7621x; 3.7621x over previous
import functools

import jax
import jax.numpy as jnp
from jax import lax
from jax.experimental import pallas as pl
from jax.experimental.pallas import tpu as pltpu
from jax.experimental.pallas import tpu_sc as plsc

B = 16384
D = 32
NW = 32
BPW = B // NW
NB = 16
NG = BPW // NB
G = 128
_I16 = lambda: lax.iota(jnp.int32, 16)


def _table_pass(t_hbm, out, idx_v, blk, colT, sem, base):
    def body(g, carry):
        iv = idx_v[pl.ds(g * NB, NB)]
        for j in range(NB):
            va = pl.multiple_of((iv[j] // G) * G, G)
            pltpu.async_copy(t_hbm.at[:, pl.ds(va, G)], blk.at[j], sem)
        for j in range(NB):
            pltpu.make_async_copy(
                t_hbm.at[:, pl.ds(0, G)], blk.at[j], sem).wait()
        lane = jax.lax.rem(iv, G)
        jvec = _I16()
        for f in range(D):
            fvec = jnp.zeros((16,), jnp.int32) + f
            vals = plsc.load_gather(blk, [jvec, fvec, lane])
            colT[f, pl.ds(g * NB, NB)] = vals
        return carry

    lax.fori_loop(0, NG, body, 0)
    pltpu.sync_copy(colT, out.at[:, pl.ds(base, BPW)])


def _gather_body(ut_hbm, it_hbm, iu_hbm, ii_hbm, u_out, i_out,
                 idx_v, blk, colT, sem):
    wid = lax.axis_index("s") * 2 + lax.axis_index("c")
    base = wid * BPW
    pltpu.sync_copy(iu_hbm.at[wid], idx_v)
    _table_pass(ut_hbm, u_out, idx_v, blk, colT, sem, base)
    pltpu.sync_copy(ii_hbm.at[wid], idx_v)
    _table_pass(it_hbm, i_out, idx_v, blk, colT, sem, base)


@jax.jit
def _sc_gather(utT, itT, idx_u2, idx_i2):
    mesh = plsc.VectorSubcoreMesh(core_axis_name="c", subcore_axis_name="s")
    out = jax.ShapeDtypeStruct((D, B), jnp.float32)
    return pl.kernel(
        _gather_body,
        mesh=mesh,
        compiler_params=pltpu.CompilerParams(
            needs_layout_passes=False, use_tc_tiling_on_sc=True),
        out_type=(out, out),
        scratch_types=[
            pltpu.VMEM((BPW,), jnp.int32),
            pltpu.VMEM((NB, D, G), jnp.float32),
            pltpu.VMEM((D, BPW), jnp.float32),
            pltpu.SemaphoreType.DMA,
        ],
    )(utT, itT, idx_u2, idx_i2)


TB = 4096


def _mlp_body(u_ref, i_ref, w1u_ref, w1i_ref, b1_ref, w2_ref, b2_ref,
              w3_ref, b3_ref, o_ref):
    h = jnp.dot(w1u_ref[...], u_ref[...], preferred_element_type=jnp.float32)
    h = h + jnp.dot(w1i_ref[...], i_ref[...], preferred_element_type=jnp.float32)
    h = jax.nn.relu(h + b1_ref[...])
    h = jax.nn.relu(
        jnp.dot(w2_ref[...], h, preferred_element_type=jnp.float32) + b2_ref[...])
    o_ref[...] = (
        jnp.dot(w3_ref[...], h, preferred_element_type=jnp.float32) + b3_ref[...])


@jax.jit
def _tc_mlp(uT, iT, W1uT, W1iT, b1, W2T, b2, W3T, b3):
    full = lambda r, c: pl.BlockSpec((r, c), lambda i: (0, 0))
    return pl.pallas_call(
        _mlp_body,
        grid=(B // TB,),
        in_specs=[
            pl.BlockSpec((D, TB), lambda i: (0, i)),
            pl.BlockSpec((D, TB), lambda i: (0, i)),
            full(64, D), full(64, D), full(64, 1),
            full(16, 64), full(16, 1),
            full(1, 16), full(1, 1),
        ],
        out_specs=pl.BlockSpec((1, TB), lambda i: (0, i)),
        out_shape=jax.ShapeDtypeStruct((1, B), jnp.float32),
    )(uT, iT, W1uT, W1iT, b1, W2T, b2, W3T, b3)


def kernel(inputs, user_table, item_table, W1, b1, W2, b2, W3, b3):
    idx_u2 = inputs[:, 0].reshape(NW, BPW)
    idx_i2 = inputs[:, 1].reshape(NW, BPW)
    uT, iT = _sc_gather(user_table.T, item_table.T, idx_u2, idx_i2)
    outT = _tc_mlp(
        uT, iT,
        W1[:D, :].T, W1[D:, :].T, b1.reshape(64, 1),
        W2.T, b2.reshape(16, 1),
        W3.T, b3.reshape(1, 1),
    )
    return outT.reshape(B, 1)


# speedup vs baseline: 4.0537x; 1.0775x over previous
import functools

import jax
import jax.numpy as jnp
from jax import lax
from jax.experimental import pallas as pl
from jax.experimental.pallas import tpu as pltpu
from jax.experimental.pallas import tpu_sc as plsc

B = 16384
D = 32
NW = 32
BPW = B // NW
G = 128
MAXQ = 24
_I16 = lambda: lax.iota(jnp.int32, 16)


def _table_pass(t_hbm, out, idx_v, blk, colT, sems, base):
    def issue(s):
        ivv = plsc.load_gather(idx_v, [jnp.zeros((16,), jnp.int32) + s])
        va = pl.multiple_of((ivv[0] // G) * G, G)
        r = lax.rem(s, MAXQ)
        pltpu.async_copy(t_hbm.at[:, pl.ds(va, G)], blk.at[r], sems.at[r])

    for s0 in range(MAXQ):
        issue(s0)

    def body(s, carry):
        r = lax.rem(s, MAXQ)
        pltpu.make_async_copy(
            t_hbm.at[:, pl.ds(0, G)], blk.at[r], sems.at[r]).wait()
        ivv = plsc.load_gather(idx_v, [jnp.zeros((16,), jnp.int32) + s])
        lane = lax.rem(ivv, G)
        rvec = jnp.zeros((16,), jnp.int32) + r
        svec = jnp.zeros((16,), jnp.int32) + s
        f0 = _I16()
        f1 = f0 + 16
        v0 = plsc.load_gather(blk, [rvec, f0, lane])
        v1 = plsc.load_gather(blk, [rvec, f1, lane])
        plsc.store_scatter(colT, [f0, svec], v0)
        plsc.store_scatter(colT, [f1, svec], v1)

        @pl.when(s + MAXQ < BPW)
        def _():
            issue(s + MAXQ)

        return carry

    lax.fori_loop(0, BPW, body, 0)
    pltpu.sync_copy(colT, out.at[:, pl.ds(base, BPW)])


def _gather_body(ut_hbm, it_hbm, iu_hbm, ii_hbm, u_out, i_out,
                 idx_v, blk, colT, sems):
    wid = lax.axis_index("s") * 2 + lax.axis_index("c")
    base = wid * BPW
    pltpu.sync_copy(iu_hbm.at[wid], idx_v)
    _table_pass(ut_hbm, u_out, idx_v, blk, colT, sems, base)
    pltpu.sync_copy(ii_hbm.at[wid], idx_v)
    _table_pass(it_hbm, i_out, idx_v, blk, colT, sems, base)


@jax.jit
def _sc_gather(utT, itT, idx_u2, idx_i2):
    mesh = plsc.VectorSubcoreMesh(core_axis_name="c", subcore_axis_name="s")
    out = jax.ShapeDtypeStruct((D, B), jnp.float32)
    return pl.kernel(
        _gather_body,
        mesh=mesh,
        compiler_params=pltpu.CompilerParams(
            needs_layout_passes=False, use_tc_tiling_on_sc=True),
        out_type=(out, out),
        scratch_types=[
            pltpu.VMEM((BPW,), jnp.int32),
            pltpu.VMEM((MAXQ, D, G), jnp.float32),
            pltpu.VMEM((D, BPW), jnp.float32),
            pltpu.SemaphoreType.DMA((MAXQ,)),
        ],
    )(utT, itT, idx_u2, idx_i2)


TB = 4096


def _mlp_body(u_ref, i_ref, w1u_ref, w1i_ref, b1_ref, w2_ref, b2_ref,
              w3_ref, b3_ref, o_ref):
    h = jnp.dot(w1u_ref[...], u_ref[...], preferred_element_type=jnp.float32)
    h = h + jnp.dot(w1i_ref[...], i_ref[...], preferred_element_type=jnp.float32)
    h = jax.nn.relu(h + b1_ref[...])
    h = jax.nn.relu(
        jnp.dot(w2_ref[...], h, preferred_element_type=jnp.float32) + b2_ref[...])
    o_ref[...] = (
        jnp.dot(w3_ref[...], h, preferred_element_type=jnp.float32) + b3_ref[...])


@jax.jit
def _tc_mlp(uT, iT, W1uT, W1iT, b1, W2T, b2, W3T, b3):
    full = lambda r, c: pl.BlockSpec((r, c), lambda i: (0, 0))
    return pl.pallas_call(
        _mlp_body,
        grid=(B // TB,),
        in_specs=[
            pl.BlockSpec((D, TB), lambda i: (0, i)),
            pl.BlockSpec((D, TB), lambda i: (0, i)),
            full(64, D), full(64, D), full(64, 1),
            full(16, 64), full(16, 1),
            full(1, 16), full(1, 1),
        ],
        out_specs=pl.BlockSpec((1, TB), lambda i: (0, i)),
        out_shape=jax.ShapeDtypeStruct((1, B), jnp.float32),
    )(uT, iT, W1uT, W1iT, b1, W2T, b2, W3T, b3)


def kernel(inputs, user_table, item_table, W1, b1, W2, b2, W3, b3):
    idx_u2 = inputs[:, 0].reshape(NW, BPW)
    idx_i2 = inputs[:, 1].reshape(NW, BPW)
    uT, iT = _sc_gather(user_table.T, item_table.T, idx_u2, idx_i2)
    outT = _tc_mlp(
        uT, iT,
        W1[:D, :].T, W1[D:, :].T, b1.reshape(64, 1),
        W2.T, b2.reshape(16, 1),
        W3.T, b3.reshape(1, 1),
    )
    return outT.reshape(B, 1)


# speedup vs baseline: 4.0698x; 1.0040x over previous
import functools

import jax
import jax.numpy as jnp
from jax import lax
from jax.experimental import pallas as pl
from jax.experimental.pallas import tpu as pltpu
from jax.experimental.pallas import tpu_sc as plsc

B = 16384
D = 32
NW = 32
BPW = B // NW
G = 128
MAXQ = 26
_I16 = lambda: lax.iota(jnp.int32, 16)


def _table_pass(t_hbm, out, idx_v, blk, colT, sems, base):
    def issue(s):
        ivv = plsc.load_gather(idx_v, [jnp.zeros((16,), jnp.int32) + s])
        va = pl.multiple_of((ivv[0] // G) * G, G)
        r = lax.rem(s, MAXQ)
        pltpu.async_copy(t_hbm.at[:, pl.ds(va, G)], blk.at[r], sems.at[r])

    for s0 in range(MAXQ):
        issue(s0)

    def body(s, carry):
        r = lax.rem(s, MAXQ)
        pltpu.make_async_copy(
            t_hbm.at[:, pl.ds(0, G)], blk.at[r], sems.at[r]).wait()
        ivv = plsc.load_gather(idx_v, [jnp.zeros((16,), jnp.int32) + s])
        lane = lax.rem(ivv, G)
        rvec = jnp.zeros((16,), jnp.int32) + r
        svec = jnp.zeros((16,), jnp.int32) + s
        f0 = _I16()
        f1 = f0 + 16
        v0 = plsc.load_gather(blk, [rvec, f0, lane])
        v1 = plsc.load_gather(blk, [rvec, f1, lane])
        plsc.store_scatter(colT, [f0, svec], v0)
        plsc.store_scatter(colT, [f1, svec], v1)

        @pl.when(s + MAXQ < BPW)
        def _():
            issue(s + MAXQ)

        return carry

    lax.fori_loop(0, BPW, body, 0)
    pltpu.sync_copy(colT, out.at[:, pl.ds(base, BPW)])


def _gather_body(ut_hbm, it_hbm, iu_hbm, ii_hbm, u_out, i_out,
                 idx_v, blk, colT, sems):
    wid = lax.axis_index("s") * 2 + lax.axis_index("c")
    base = wid * BPW
    pltpu.sync_copy(iu_hbm.at[wid], idx_v)
    _table_pass(ut_hbm, u_out, idx_v, blk, colT, sems, base)
    pltpu.sync_copy(ii_hbm.at[wid], idx_v)
    _table_pass(it_hbm, i_out, idx_v, blk, colT, sems, base)


@jax.jit
def _sc_gather(utT, itT, idx_u2, idx_i2):
    mesh = plsc.VectorSubcoreMesh(core_axis_name="c", subcore_axis_name="s")
    out = jax.ShapeDtypeStruct((D, B), jnp.float32)
    return pl.kernel(
        _gather_body,
        mesh=mesh,
        compiler_params=pltpu.CompilerParams(
            needs_layout_passes=False, use_tc_tiling_on_sc=True),
        out_type=(out, out),
        scratch_types=[
            pltpu.VMEM((BPW,), jnp.int32),
            pltpu.VMEM((MAXQ, D, G), jnp.float32),
            pltpu.VMEM((D, BPW), jnp.float32),
            pltpu.SemaphoreType.DMA((MAXQ,)),
        ],
    )(utT, itT, idx_u2, idx_i2)


TB = 4096


def _mlp_body(u_ref, i_ref, w1u_ref, w1i_ref, b1_ref, w2_ref, b2_ref,
              w3_ref, b3_ref, o_ref):
    h = jnp.dot(w1u_ref[...], u_ref[...], preferred_element_type=jnp.float32)
    h = h + jnp.dot(w1i_ref[...], i_ref[...], preferred_element_type=jnp.float32)
    h = jax.nn.relu(h + b1_ref[...])
    h = jax.nn.relu(
        jnp.dot(w2_ref[...], h, preferred_element_type=jnp.float32) + b2_ref[...])
    o_ref[...] = (
        jnp.dot(w3_ref[...], h, preferred_element_type=jnp.float32) + b3_ref[...])


@jax.jit
def _tc_mlp(uT, iT, W1uT, W1iT, b1, W2T, b2, W3T, b3):
    full = lambda r, c: pl.BlockSpec((r, c), lambda i: (0, 0))
    return pl.pallas_call(
        _mlp_body,
        grid=(B // TB,),
        in_specs=[
            pl.BlockSpec((D, TB), lambda i: (0, i)),
            pl.BlockSpec((D, TB), lambda i: (0, i)),
            full(64, D), full(64, D), full(64, 1),
            full(16, 64), full(16, 1),
            full(1, 16), full(1, 1),
        ],
        out_specs=pl.BlockSpec((1, TB), lambda i: (0, i)),
        out_shape=jax.ShapeDtypeStruct((1, B), jnp.float32),
    )(uT, iT, W1uT, W1iT, b1, W2T, b2, W3T, b3)


def kernel(inputs, user_table, item_table, W1, b1, W2, b2, W3, b3):
    idx_u2 = inputs[:, 0].reshape(NW, BPW)
    idx_i2 = inputs[:, 1].reshape(NW, BPW)
    uT, iT = _sc_gather(user_table.T, item_table.T, idx_u2, idx_i2)
    outT = _tc_mlp(
        uT, iT,
        W1[:D, :].T, W1[D:, :].T, b1.reshape(64, 1),
        W2.T, b2.reshape(16, 1),
        W3.T, b3.reshape(1, 1),
    )
    return outT.reshape(B, 1)


# speedup vs baseline: 4.0794x; 1.0024x over previous
import functools

import jax
import jax.numpy as jnp
from jax import lax
from jax.experimental import pallas as pl
from jax.experimental.pallas import tpu as pltpu
from jax.experimental.pallas import tpu_sc as plsc

B = 16384
D = 32
NW = 32
BPW = B // NW
G = 128
K = 11
_I16 = lambda: lax.iota(jnp.int32, 16)


def _issue(t_hbm, idx_v, blk, sems, s, slot):
    ivv = plsc.load_gather(idx_v, [jnp.zeros((16,), jnp.int32) + s])
    va = pl.multiple_of((ivv[0] // G) * G, G)
    pltpu.async_copy(t_hbm.at[:, pl.ds(va, G)], blk.at[slot], sems.at[slot])


def _extract(dummy_hbm, idx_v, blk, sems, col, s, slot):
    pltpu.make_async_copy(
        dummy_hbm.at[:, pl.ds(0, G)], blk.at[slot], sems.at[slot]).wait()
    ivv = plsc.load_gather(idx_v, [jnp.zeros((16,), jnp.int32) + s])
    lane = lax.rem(ivv, G)
    svec = jnp.zeros((16,), jnp.int32) + s
    rvec = jnp.zeros((16,), jnp.int32) + slot
    f0 = _I16()
    f1 = f0 + 16
    v0 = plsc.load_gather(blk, [rvec, f0, lane])
    v1 = plsc.load_gather(blk, [rvec, f1, lane])
    plsc.store_scatter(col, [f0, svec], v0)
    plsc.store_scatter(col, [f1, svec], v1)


def _gather_body(ut_hbm, it_hbm, iu_hbm, ii_hbm, u_out, i_out,
                 idxu_v, idxi_v, blk, colU, colI, sems):
    wid = lax.axis_index("s") * 2 + lax.axis_index("c")
    base = wid * BPW
    pltpu.sync_copy(iu_hbm.at[wid], idxu_v)
    pltpu.sync_copy(ii_hbm.at[wid], idxi_v)

    for s0 in range(K):
        _issue(ut_hbm, idxu_v, blk, sems, s0, 2 * s0)
        _issue(it_hbm, idxi_v, blk, sems, s0, 2 * s0 + 1)

    def body(s, carry):
        r = lax.rem(s, K)
        _extract(ut_hbm, idxu_v, blk, sems, colU, s, 2 * r)
        _extract(ut_hbm, idxi_v, blk, sems, colI, s, 2 * r + 1)

        @pl.when(s + K < BPW)
        def _():
            _issue(ut_hbm, idxu_v, blk, sems, s + K, 2 * r)
            _issue(it_hbm, idxi_v, blk, sems, s + K, 2 * r + 1)

        return carry

    lax.fori_loop(0, BPW, body, 0)
    pltpu.sync_copy(colU, u_out.at[:, pl.ds(base, BPW)])
    pltpu.sync_copy(colI, i_out.at[:, pl.ds(base, BPW)])


@jax.jit
def _sc_gather(utT, itT, idx_u2, idx_i2):
    mesh = plsc.VectorSubcoreMesh(core_axis_name="c", subcore_axis_name="s")
    out = jax.ShapeDtypeStruct((D, B), jnp.float32)
    return pl.kernel(
        _gather_body,
        mesh=mesh,
        compiler_params=pltpu.CompilerParams(
            needs_layout_passes=False, use_tc_tiling_on_sc=True),
        out_type=(out, out),
        scratch_types=[
            pltpu.VMEM((BPW,), jnp.int32),
            pltpu.VMEM((BPW,), jnp.int32),
            pltpu.VMEM((2 * K, D, G), jnp.float32),
            pltpu.VMEM((D, BPW), jnp.float32),
            pltpu.VMEM((D, BPW), jnp.float32),
            pltpu.SemaphoreType.DMA((2 * K,)),
        ],
    )(utT, itT, idx_u2, idx_i2)


TB = 4096


def _mlp_body(u_ref, i_ref, w1u_ref, w1i_ref, b1_ref, w2_ref, b2_ref,
              w3_ref, b3_ref, o_ref):
    h = jnp.dot(w1u_ref[...], u_ref[...], preferred_element_type=jnp.float32)
    h = h + jnp.dot(w1i_ref[...], i_ref[...], preferred_element_type=jnp.float32)
    h = jax.nn.relu(h + b1_ref[...])
    h = jax.nn.relu(
        jnp.dot(w2_ref[...], h, preferred_element_type=jnp.float32) + b2_ref[...])
    o_ref[...] = (
        jnp.dot(w3_ref[...], h, preferred_element_type=jnp.float32) + b3_ref[...])


@jax.jit
def _tc_mlp(uT, iT, W1uT, W1iT, b1, W2T, b2, W3T, b3):
    full = lambda r, c: pl.BlockSpec((r, c), lambda i: (0, 0))
    return pl.pallas_call(
        _mlp_body,
        grid=(B // TB,),
        in_specs=[
            pl.BlockSpec((D, TB), lambda i: (0, i)),
            pl.BlockSpec((D, TB), lambda i: (0, i)),
            full(64, D), full(64, D), full(64, 1),
            full(16, 64), full(16, 1),
            full(1, 16), full(1, 1),
        ],
        out_specs=pl.BlockSpec((1, TB), lambda i: (0, i)),
        out_shape=jax.ShapeDtypeStruct((1, B), jnp.float32),
    )(uT, iT, W1uT, W1iT, b1, W2T, b2, W3T, b3)


def kernel(inputs, user_table, item_table, W1, b1, W2, b2, W3, b3):
    idx_u2 = inputs[:, 0].reshape(NW, BPW)
    idx_i2 = inputs[:, 1].reshape(NW, BPW)
    uT, iT = _sc_gather(user_table.T, item_table.T, idx_u2, idx_i2)
    outT = _tc_mlp(
        uT, iT,
        W1[:D, :].T, W1[D:, :].T, b1.reshape(64, 1),
        W2.T, b2.reshape(16, 1),
        W3.T, b3.reshape(1, 1),
    )
    return outT.reshape(B, 1)
